# baseline (speedup 1.0000x reference)
import functools

import jax
import jax.numpy as jnp
from jax import lax
from jax.experimental import pallas as pl
from jax.experimental.pallas import tpu as pltpu
from jax.experimental.pallas import tpu_sc as plsc

_N = 10000
_E = 320000
_D = 128
_CB = 50
_NC = 2
_NS = 16
_NW = _NC * _NS
_ROWS_W = _E // (_CB * _NW)
_SBB = 20
_NBUF = 5
_LA = _NBUF - 1
_NSB = _ROWS_W // _SBB
_PSB = _SBB // 2
_NPT = 624
_NTAIL = _N - _NS * _NPT


def _sc_mesh():
    return plsc.VectorSubcoreMesh(core_axis_name="c", subcore_axis_name="s")


def _copy_node_rows(src_ref, dst_ref, s):
    pltpu.sync_copy(src_ref.at[pl.ds(s * _NPT, _NPT)],
                    dst_ref.at[pl.ds(s * _NPT, _NPT)])

    @pl.when(s == _NS - 1)
    def _():
        pltpu.sync_copy(src_ref.at[pl.ds(_NS * _NPT, _NTAIL)],
                        dst_ref.at[pl.ds(_NS * _NPT, _NTAIL)])


@functools.partial(
    pl.kernel,
    out_type=jax.ShapeDtypeStruct((_NC, _N), jnp.float32),
    scratch_types=[
        pltpu.VMEM_SHARED((_N,), jnp.float32),
        pltpu.VMEM((_ROWS_W, _CB), jnp.int32),
        pltpu.VMEM((_CB,), jnp.float32),
    ],
    mesh=_sc_mesh(),
)
def _deg_kernel(ones_hbm, dst_hbm, out_hbm, deg_sh, dst_blk, ones_v):
    c = lax.axis_index("c")
    s = lax.axis_index("s")
    w = c * _NS + s

    @pl.when(s == 0)
    def _():
        pltpu.sync_copy(ones_hbm, deg_sh)

    pltpu.sync_copy(dst_hbm.at[w], dst_blk)
    pltpu.sync_copy(ones_hbm.at[pl.ds(0, _CB)], ones_v)
    plsc.subcore_barrier()

    def body(j, carry):
        pltpu.sync_copy(ones_v, deg_sh.at[dst_blk.at[j]], add=True)
        return carry

    lax.fori_loop(0, _ROWS_W, body, 0)
    plsc.subcore_barrier()

    @pl.when(s == 0)
    def _():
        pltpu.sync_copy(deg_sh, out_hbm.at[c])


@functools.partial(
    pl.kernel,
    out_type=jax.ShapeDtypeStruct((_NC, _N, _D), jnp.float32),
    scratch_types=[
        pltpu.VMEM_SHARED((_N, _D), jnp.float32),
        pltpu.VMEM((2, _SBB, _CB), jnp.int32),
        pltpu.VMEM((2, _SBB, _CB), jnp.int32),
        pltpu.VMEM((_NBUF, _CB, _D), jnp.float32),
        pltpu.SemaphoreType.DMA,
        pltpu.SemaphoreType.DMA,
        pltpu.SemaphoreType.DMA,
    ],
    mesh=_sc_mesh(),
)
def _agg_kernel(g_hbm, src_hbm, dst_hbm, out_hbm, acc_sh, srcb, dstb,
                rows, gsem, ssem, isem):
    c = lax.axis_index("c")
    s = lax.axis_index("s")
    w = c * _NS + s
    src_w = src_hbm.at[w]
    dst_w = dst_hbm.at[w]
    nblk = _NSB * _SBB

    _copy_node_rows(g_hbm, acc_sh, s)
    pltpu.sync_copy(src_w.at[0], srcb.at[0])
    pltpu.sync_copy(dst_w.at[0], dstb.at[0])
    pltpu.async_copy(src_w.at[1], srcb.at[1], isem)
    pltpu.async_copy(dst_w.at[1], dstb.at[1], isem)
    plsc.subcore_barrier()

    def _gather_start(q, r, m):
        pltpu.async_copy(g_hbm.at[srcb.at[q, r]], rows.at[m], gsem)

    def _scatter_start(q, r, m):
        pltpu.async_copy(rows.at[m], acc_sh.at[dstb.at[q, r]], ssem, add=True)

    def _wait_rows(sem):
        pltpu.make_async_copy(g_hbm.at[srcb.at[0, 0]], rows.at[0], sem).wait()

    for i in range(_LA):
        _gather_start(0, i, i)

    def body(j, carry):
        sb = j // _SBB
        r = lax.rem(j, _SBB)
        q = lax.rem(sb, 2)
        m = lax.rem(j, _NBUF)
        j2 = j + _LA
        sb2 = j2 // _SBB
        r2 = lax.rem(j2, _SBB)
        q2 = lax.rem(sb2, 2)
        m2 = lax.rem(j2, _NBUF)

        _wait_rows(gsem)
        _scatter_start(q, r, m)

        @pl.when(j >= 1)
        def _():
            _wait_rows(ssem)

        @pl.when(jnp.logical_and(r == 2, jnp.logical_and(sb >= 1, sb + 1 < _NSB)))
        def _():
            pltpu.async_copy(src_w.at[sb + 1], srcb.at[1 - q], isem)
            pltpu.async_copy(dst_w.at[sb + 1], dstb.at[1 - q], isem)

        @pl.when(jnp.logical_and(r == _SBB - _LA, sb + 1 < _NSB))
        def _():
            pltpu.make_async_copy(src_w.at[0], srcb.at[0], isem).wait()
            pltpu.make_async_copy(dst_w.at[0], dstb.at[0], isem).wait()

        @pl.when(j2 < nblk)
        def _():
            _gather_start(q2, r2, m2)

        return carry

    lax.fori_loop(0, nblk, body, 0)
    _wait_rows(ssem)
    plsc.subcore_barrier()

    _copy_node_rows(acc_sh, out_hbm.at[c], s)


_BN = 2000


def _t1_body(d0, d1, x, w, o):
    dinv = lax.rsqrt(d0[...] + d1[...] - 1.0)
    o[...] = dinv * jnp.dot(x[...], w[...], preferred_element_type=jnp.float32)


def _t2_body(d0, d1, a0, a1, g, b, w, o):
    dinv = lax.rsqrt(d0[...] + d1[...] - 1.0)
    u = jnp.maximum(dinv * (a0[...] + a1[...] - g[...]) + b[...], 0.0)
    o[...] = dinv * jnp.dot(u, w[...], preferred_element_type=jnp.float32)


def _t3_body(d0, d1, a0, a1, g, b, o):
    dinv = lax.rsqrt(d0[...] + d1[...] - 1.0)
    o[...] = jnp.maximum(dinv * (a0[...] + a1[...] - g[...]) + b[...], 0.0)


_col = pl.BlockSpec((_BN, 1), lambda i: (i, 0))
_mat = pl.BlockSpec((_BN, _D), lambda i: (i, 0))
_wspec = pl.BlockSpec((_D, _D), lambda i: (0, 0))
_bspec = pl.BlockSpec((1, _D), lambda i: (0, 0))
_oshape = jax.ShapeDtypeStruct((_N, _D), jnp.float32)


def _t1(d0, d1, x, W):
    return pl.pallas_call(
        _t1_body,
        grid=(_N // _BN,),
        in_specs=[_col, _col, _mat, _wspec],
        out_specs=_mat,
        out_shape=_oshape,
    )(d0, d1, x, W)


def _t2(d0, d1, a0, a1, g, b, W):
    return pl.pallas_call(
        _t2_body,
        grid=(_N // _BN,),
        in_specs=[_col, _col, _mat, _mat, _mat, _bspec, _wspec],
        out_specs=_mat,
        out_shape=_oshape,
    )(d0, d1, a0, a1, g, b, W)


def _t3(d0, d1, a0, a1, g, b):
    return pl.pallas_call(
        _t3_body,
        grid=(_N // _BN,),
        in_specs=[_col, _col, _mat, _mat, _mat, _bspec],
        out_specs=_mat,
        out_shape=_oshape,
    )(d0, d1, a0, a1, g, b)


def kernel(x, edge_index, W1, b1, W2, b2):
    src4 = edge_index[0].reshape(_NW, _NSB, _SBB, _CB)
    dst4 = edge_index[1].reshape(_NW, _NSB, _SBB, _CB)
    dst2 = edge_index[1].reshape(_NW, _ROWS_W, _CB)
    ones1 = jnp.ones((_N,), jnp.float32)

    degp = _deg_kernel(ones1, dst2)
    d0 = degp[0].reshape(_N, 1)
    d1 = degp[1].reshape(_N, 1)

    g1 = _t1(d0, d1, x, W1)
    acc = _agg_kernel(g1, src4, dst4)
    g2 = _t2(d0, d1, acc[0], acc[1], g1, b1.reshape(1, _D), W2)
    acc2 = _agg_kernel(g2, src4, dst4)
    return _t3(d0, d1, acc2[0], acc2[1], g2, b2.reshape(1, _D))


# speedup vs baseline: 1.0358x; 1.0358x over previous
import functools

import jax
import jax.numpy as jnp
from jax import lax
from jax.experimental import pallas as pl
from jax.experimental.pallas import tpu as pltpu
from jax.experimental.pallas import tpu_sc as plsc

_N = 10000
_E = 320000
_D = 128
_CB = 100
_NC = 2
_NS = 16
_NW = _NC * _NS
_ROWS_W = _E // (_CB * _NW)
_SBB = 10
_NSB = _ROWS_W // _SBB
_PSB = _SBB // 2
_NPT = 624
_NTAIL = _N - _NS * _NPT


def _sc_mesh():
    return plsc.VectorSubcoreMesh(core_axis_name="c", subcore_axis_name="s")


def _copy_node_rows(src_ref, dst_ref, s):
    pltpu.sync_copy(src_ref.at[pl.ds(s * _NPT, _NPT)],
                    dst_ref.at[pl.ds(s * _NPT, _NPT)])

    @pl.when(s == _NS - 1)
    def _():
        pltpu.sync_copy(src_ref.at[pl.ds(_NS * _NPT, _NTAIL)],
                        dst_ref.at[pl.ds(_NS * _NPT, _NTAIL)])


@functools.partial(
    pl.kernel,
    out_type=jax.ShapeDtypeStruct((_NC, _N), jnp.float32),
    scratch_types=[
        pltpu.VMEM_SHARED((_N,), jnp.float32),
        pltpu.VMEM((_ROWS_W, _CB), jnp.int32),
        pltpu.VMEM((_CB,), jnp.float32),
    ],
    mesh=_sc_mesh(),
)
def _deg_kernel(ones_hbm, dst_hbm, out_hbm, deg_sh, dst_blk, ones_v):
    c = lax.axis_index("c")
    s = lax.axis_index("s")
    w = c * _NS + s

    @pl.when(s == 0)
    def _():
        pltpu.sync_copy(ones_hbm, deg_sh)

    pltpu.sync_copy(dst_hbm.at[w], dst_blk)
    pltpu.sync_copy(ones_hbm.at[pl.ds(0, _CB)], ones_v)
    plsc.subcore_barrier()

    def body(j, carry):
        pltpu.sync_copy(ones_v, deg_sh.at[dst_blk.at[j]], add=True)
        return carry

    lax.fori_loop(0, _ROWS_W, body, 0)
    plsc.subcore_barrier()

    @pl.when(s == 0)
    def _():
        pltpu.sync_copy(deg_sh, out_hbm.at[c])


@functools.partial(
    pl.kernel,
    out_type=jax.ShapeDtypeStruct((_NC, _N, _D), jnp.float32),
    scratch_types=[
        pltpu.VMEM_SHARED((_N, _D), jnp.float32),
        pltpu.VMEM((2, _SBB, _CB), jnp.int32),
        pltpu.VMEM((2, _SBB, _CB), jnp.int32),
        pltpu.VMEM((3, _CB, _D), jnp.float32),
        pltpu.SemaphoreType.DMA,
        pltpu.SemaphoreType.DMA,
        pltpu.SemaphoreType.DMA,
        pltpu.SemaphoreType.DMA,
    ],
    mesh=_sc_mesh(),
)
def _agg_kernel(g_hbm, src_hbm, dst_hbm, out_hbm, acc_sh, srcb, dstb,
                rows, gsem, ssem, isem, nsem):
    c = lax.axis_index("c")
    s = lax.axis_index("s")
    w = c * _NS + s
    src_w = src_hbm.at[w]
    dst_w = dst_hbm.at[w]
    nblk = _NSB * _SBB

    def _gather_start(q, r, m):
        pltpu.async_copy(g_hbm.at[srcb.at[q, r]], rows.at[m], gsem)

    def _scatter_start(q, r, m):
        pltpu.async_copy(rows.at[m], acc_sh.at[dstb.at[q, r]], ssem, add=True)

    def _wait_rows(sem):
        pltpu.make_async_copy(g_hbm.at[srcb.at[0, 0]], rows.at[0], sem).wait()

    _init = pltpu.async_copy(g_hbm.at[pl.ds(s * _NPT, _NPT)],
                             acc_sh.at[pl.ds(s * _NPT, _NPT)], nsem)

    @pl.when(s == _NS - 1)
    def _():
        pltpu.async_copy(g_hbm.at[pl.ds(_NS * _NPT, _NTAIL)],
                         acc_sh.at[pl.ds(_NS * _NPT, _NTAIL)], nsem)

    pltpu.sync_copy(src_w.at[0], srcb.at[0])
    pltpu.sync_copy(dst_w.at[0], dstb.at[0])
    pltpu.async_copy(src_w.at[1], srcb.at[1], isem)
    pltpu.async_copy(dst_w.at[1], dstb.at[1], isem)

    _gather_start(0, 0, 0)
    _gather_start(0, 1, 1)

    pltpu.make_async_copy(g_hbm.at[pl.ds(s * _NPT, _NPT)],
                          acc_sh.at[pl.ds(s * _NPT, _NPT)], nsem).wait()

    @pl.when(s == _NS - 1)
    def _():
        pltpu.make_async_copy(g_hbm.at[pl.ds(_NS * _NPT, _NTAIL)],
                              acc_sh.at[pl.ds(_NS * _NPT, _NTAIL)], nsem).wait()

    plsc.subcore_barrier()

    def body(j, carry):
        sb = j // _SBB
        r = lax.rem(j, _SBB)
        q = lax.rem(sb, 2)
        m = lax.rem(j, 3)
        j2 = j + 2
        sb2 = j2 // _SBB
        r2 = lax.rem(j2, _SBB)
        q2 = lax.rem(sb2, 2)
        m2 = lax.rem(j2, 3)

        _wait_rows(gsem)
        _scatter_start(q, r, m)

        @pl.when(j >= 1)
        def _():
            _wait_rows(ssem)

        @pl.when(jnp.logical_and(r == 2, jnp.logical_and(sb >= 1, sb + 1 < _NSB)))
        def _():
            pltpu.async_copy(src_w.at[sb + 1], srcb.at[1 - q], isem)
            pltpu.async_copy(dst_w.at[sb + 1], dstb.at[1 - q], isem)

        @pl.when(jnp.logical_and(r == _SBB - 2, sb + 1 < _NSB))
        def _():
            pltpu.make_async_copy(src_w.at[0], srcb.at[0], isem).wait()
            pltpu.make_async_copy(dst_w.at[0], dstb.at[0], isem).wait()

        @pl.when(j2 < nblk)
        def _():
            _gather_start(q2, r2, m2)

        return carry

    lax.fori_loop(0, nblk, body, 0)
    _wait_rows(ssem)
    plsc.subcore_barrier()

    _copy_node_rows(acc_sh, out_hbm.at[c], s)


_BN = 2000


def _t1_body(d0, d1, x, w, o):
    dinv = lax.rsqrt(d0[...] + d1[...] - 1.0)
    o[...] = dinv * jnp.dot(x[...], w[...], preferred_element_type=jnp.float32)


def _t2_body(d0, d1, a0, a1, g, b, w, o):
    dinv = lax.rsqrt(d0[...] + d1[...] - 1.0)
    u = jnp.maximum(dinv * (a0[...] + a1[...] - g[...]) + b[...], 0.0)
    o[...] = dinv * jnp.dot(u, w[...], preferred_element_type=jnp.float32)


def _t3_body(d0, d1, a0, a1, g, b, o):
    dinv = lax.rsqrt(d0[...] + d1[...] - 1.0)
    o[...] = jnp.maximum(dinv * (a0[...] + a1[...] - g[...]) + b[...], 0.0)


_col = pl.BlockSpec((_BN, 1), lambda i: (i, 0))
_mat = pl.BlockSpec((_BN, _D), lambda i: (i, 0))
_wspec = pl.BlockSpec((_D, _D), lambda i: (0, 0))
_bspec = pl.BlockSpec((1, _D), lambda i: (0, 0))
_oshape = jax.ShapeDtypeStruct((_N, _D), jnp.float32)


def _t1(d0, d1, x, W):
    return pl.pallas_call(
        _t1_body,
        grid=(_N // _BN,),
        in_specs=[_col, _col, _mat, _wspec],
        out_specs=_mat,
        out_shape=_oshape,
    )(d0, d1, x, W)


def _t2(d0, d1, a0, a1, g, b, W):
    return pl.pallas_call(
        _t2_body,
        grid=(_N // _BN,),
        in_specs=[_col, _col, _mat, _mat, _mat, _bspec, _wspec],
        out_specs=_mat,
        out_shape=_oshape,
    )(d0, d1, a0, a1, g, b, W)


def _t3(d0, d1, a0, a1, g, b):
    return pl.pallas_call(
        _t3_body,
        grid=(_N // _BN,),
        in_specs=[_col, _col, _mat, _mat, _mat, _bspec],
        out_specs=_mat,
        out_shape=_oshape,
    )(d0, d1, a0, a1, g, b)


def kernel(x, edge_index, W1, b1, W2, b2):
    src4 = edge_index[0].reshape(_NW, _NSB, _SBB, _CB)
    dst4 = edge_index[1].reshape(_NW, _NSB, _SBB, _CB)
    dst2 = edge_index[1].reshape(_NW, _ROWS_W, _CB)
    ones1 = jnp.ones((_N,), jnp.float32)

    degp = _deg_kernel(ones1, dst2)
    d0 = degp[0].reshape(_N, 1)
    d1 = degp[1].reshape(_N, 1)

    g1 = _t1(d0, d1, x, W1)
    acc = _agg_kernel(g1, src4, dst4)
    g2 = _t2(d0, d1, acc[0], acc[1], g1, b1.reshape(1, _D), W2)
    acc2 = _agg_kernel(g2, src4, dst4)
    return _t3(d0, d1, acc2[0], acc2[1], g2, b2.reshape(1, _D))


# speedup vs baseline: 1.0606x; 1.0239x over previous
import functools

import jax
import jax.numpy as jnp
from jax import lax
from jax.experimental import pallas as pl
from jax.experimental.pallas import tpu as pltpu
from jax.experimental.pallas import tpu_sc as plsc

_N = 10000
_E = 320000
_D = 128
_CB = 100
_NC = 2
_NS = 16
_NW = _NC * _NS
_ROWS_W = _E // (_CB * _NW)
_SBB = 10
_NSB = _ROWS_W // _SBB
_PSB = _SBB // 2
_NPT = 624
_NTAIL = _N - _NS * _NPT


def _sc_mesh():
    return plsc.VectorSubcoreMesh(core_axis_name="c", subcore_axis_name="s")


def _copy_node_rows(src_ref, dst_ref, s):
    pltpu.sync_copy(src_ref.at[pl.ds(s * _NPT, _NPT)],
                    dst_ref.at[pl.ds(s * _NPT, _NPT)])

    @pl.when(s == _NS - 1)
    def _():
        pltpu.sync_copy(src_ref.at[pl.ds(_NS * _NPT, _NTAIL)],
                        dst_ref.at[pl.ds(_NS * _NPT, _NTAIL)])


@functools.partial(
    pl.kernel,
    out_type=jax.ShapeDtypeStruct((_NC, _N), jnp.float32),
    scratch_types=[
        pltpu.VMEM_SHARED((_N,), jnp.float32),
        pltpu.VMEM((_ROWS_W, _CB), jnp.int32),
        pltpu.VMEM((_CB,), jnp.float32),
        pltpu.SemaphoreType.DMA,
    ],
    mesh=_sc_mesh(),
)
def _deg_kernel(ones_hbm, dst_hbm, out_hbm, deg_sh, dst_blk, ones_v, dsem):
    c = lax.axis_index("c")
    s = lax.axis_index("s")
    w = c * _NS + s
    win = 8

    @pl.when(s == 0)
    def _():
        pltpu.sync_copy(ones_hbm, deg_sh)

    pltpu.sync_copy(dst_hbm.at[w], dst_blk)
    pltpu.sync_copy(ones_hbm.at[pl.ds(0, _CB)], ones_v)
    plsc.subcore_barrier()

    def _wait_one():
        pltpu.make_async_copy(ones_v, deg_sh.at[dst_blk.at[0]], dsem).wait()

    def body(j, carry):
        @pl.when(j >= win)
        def _():
            _wait_one()

        pltpu.async_copy(ones_v, deg_sh.at[dst_blk.at[j]], dsem, add=True)
        return carry

    lax.fori_loop(0, _ROWS_W, body, 0)
    for _ in range(win):
        _wait_one()
    plsc.subcore_barrier()

    @pl.when(s == 0)
    def _():
        pltpu.sync_copy(deg_sh, out_hbm.at[c])


@functools.partial(
    pl.kernel,
    out_type=jax.ShapeDtypeStruct((_NC, _N, _D), jnp.float32),
    scratch_types=[
        pltpu.VMEM_SHARED((_N, _D), jnp.float32),
        pltpu.VMEM((2, _SBB, _CB), jnp.int32),
        pltpu.VMEM((2, _SBB, _CB), jnp.int32),
        pltpu.VMEM((3, _CB, _D), jnp.float32),
        pltpu.SemaphoreType.DMA,
        pltpu.SemaphoreType.DMA,
        pltpu.SemaphoreType.DMA,
        pltpu.SemaphoreType.DMA,
    ],
    mesh=_sc_mesh(),
)
def _agg_kernel(g_hbm, src_hbm, dst_hbm, out_hbm, acc_sh, srcb, dstb,
                rows, gsem, ssem, isem, nsem):
    c = lax.axis_index("c")
    s = lax.axis_index("s")
    w = c * _NS + s
    src_w = src_hbm.at[w]
    dst_w = dst_hbm.at[w]
    nblk = _NSB * _SBB

    def _gather_start(q, r, m):
        pltpu.async_copy(g_hbm.at[srcb.at[q, r]], rows.at[m], gsem)

    def _scatter_start(q, r, m):
        pltpu.async_copy(rows.at[m], acc_sh.at[dstb.at[q, r]], ssem, add=True)

    def _wait_rows(sem):
        pltpu.make_async_copy(g_hbm.at[srcb.at[0, 0]], rows.at[0], sem).wait()

    _init = pltpu.async_copy(g_hbm.at[pl.ds(s * _NPT, _NPT)],
                             acc_sh.at[pl.ds(s * _NPT, _NPT)], nsem)

    @pl.when(s == _NS - 1)
    def _():
        pltpu.async_copy(g_hbm.at[pl.ds(_NS * _NPT, _NTAIL)],
                         acc_sh.at[pl.ds(_NS * _NPT, _NTAIL)], nsem)

    pltpu.sync_copy(src_w.at[0], srcb.at[0])
    pltpu.sync_copy(dst_w.at[0], dstb.at[0])
    pltpu.async_copy(src_w.at[1], srcb.at[1], isem)
    pltpu.async_copy(dst_w.at[1], dstb.at[1], isem)

    _gather_start(0, 0, 0)
    _gather_start(0, 1, 1)

    pltpu.make_async_copy(g_hbm.at[pl.ds(s * _NPT, _NPT)],
                          acc_sh.at[pl.ds(s * _NPT, _NPT)], nsem).wait()

    @pl.when(s == _NS - 1)
    def _():
        pltpu.make_async_copy(g_hbm.at[pl.ds(_NS * _NPT, _NTAIL)],
                              acc_sh.at[pl.ds(_NS * _NPT, _NTAIL)], nsem).wait()

    plsc.subcore_barrier()

    def body(j, carry):
        sb = j // _SBB
        r = lax.rem(j, _SBB)
        q = lax.rem(sb, 2)
        m = lax.rem(j, 3)
        j2 = j + 2
        sb2 = j2 // _SBB
        r2 = lax.rem(j2, _SBB)
        q2 = lax.rem(sb2, 2)
        m2 = lax.rem(j2, 3)

        _wait_rows(gsem)
        _scatter_start(q, r, m)

        @pl.when(j >= 1)
        def _():
            _wait_rows(ssem)

        @pl.when(jnp.logical_and(r == 2, jnp.logical_and(sb >= 1, sb + 1 < _NSB)))
        def _():
            pltpu.async_copy(src_w.at[sb + 1], srcb.at[1 - q], isem)
            pltpu.async_copy(dst_w.at[sb + 1], dstb.at[1 - q], isem)

        @pl.when(jnp.logical_and(r == _SBB - 2, sb + 1 < _NSB))
        def _():
            pltpu.make_async_copy(src_w.at[0], srcb.at[0], isem).wait()
            pltpu.make_async_copy(dst_w.at[0], dstb.at[0], isem).wait()

        @pl.when(j2 < nblk)
        def _():
            _gather_start(q2, r2, m2)

        return carry

    lax.fori_loop(0, nblk, body, 0)
    _wait_rows(ssem)
    plsc.subcore_barrier()

    _copy_node_rows(acc_sh, out_hbm.at[c], s)


_BN = 2000


def _t1_body(d0, d1, x, w, o):
    dinv = lax.rsqrt(d0[...] + d1[...] - 1.0)
    o[...] = dinv * jnp.dot(x[...], w[...], preferred_element_type=jnp.float32)


def _t2_body(d0, d1, a0, a1, g, b, w, o):
    dinv = lax.rsqrt(d0[...] + d1[...] - 1.0)
    u = jnp.maximum(dinv * (a0[...] + a1[...] - g[...]) + b[...], 0.0)
    o[...] = dinv * jnp.dot(u, w[...], preferred_element_type=jnp.float32)


def _t3_body(d0, d1, a0, a1, g, b, o):
    dinv = lax.rsqrt(d0[...] + d1[...] - 1.0)
    o[...] = jnp.maximum(dinv * (a0[...] + a1[...] - g[...]) + b[...], 0.0)


_col = pl.BlockSpec((_BN, 1), lambda i: (i, 0))
_mat = pl.BlockSpec((_BN, _D), lambda i: (i, 0))
_wspec = pl.BlockSpec((_D, _D), lambda i: (0, 0))
_bspec = pl.BlockSpec((1, _D), lambda i: (0, 0))
_oshape = jax.ShapeDtypeStruct((_N, _D), jnp.float32)


def _t1(d0, d1, x, W):
    return pl.pallas_call(
        _t1_body,
        grid=(_N // _BN,),
        in_specs=[_col, _col, _mat, _wspec],
        out_specs=_mat,
        out_shape=_oshape,
    )(d0, d1, x, W)


def _t2(d0, d1, a0, a1, g, b, W):
    return pl.pallas_call(
        _t2_body,
        grid=(_N // _BN,),
        in_specs=[_col, _col, _mat, _mat, _mat, _bspec, _wspec],
        out_specs=_mat,
        out_shape=_oshape,
    )(d0, d1, a0, a1, g, b, W)


def _t3(d0, d1, a0, a1, g, b):
    return pl.pallas_call(
        _t3_body,
        grid=(_N // _BN,),
        in_specs=[_col, _col, _mat, _mat, _mat, _bspec],
        out_specs=_mat,
        out_shape=_oshape,
    )(d0, d1, a0, a1, g, b)


def kernel(x, edge_index, W1, b1, W2, b2):
    src4 = edge_index[0].reshape(_NW, _NSB, _SBB, _CB)
    dst4 = edge_index[1].reshape(_NW, _NSB, _SBB, _CB)
    dst2 = edge_index[1].reshape(_NW, _ROWS_W, _CB)
    ones1 = jnp.ones((_N,), jnp.float32)

    degp = _deg_kernel(ones1, dst2)
    d0 = degp[0].reshape(_N, 1)
    d1 = degp[1].reshape(_N, 1)

    g1 = _t1(d0, d1, x, W1)
    acc = _agg_kernel(g1, src4, dst4)
    g2 = _t2(d0, d1, acc[0], acc[1], g1, b1.reshape(1, _D), W2)
    acc2 = _agg_kernel(g2, src4, dst4)
    return _t3(d0, d1, acc2[0], acc2[1], g2, b2.reshape(1, _D))
